# Initial kernel scaffold; baseline (speedup 1.0000x reference)
import jax
import jax.numpy as jnp
from jax.experimental import pallas as pl


def kernel(x, embed):
    raise NotImplementedError("write your pallas kernel here")


# speedup vs baseline: 2.5411x; 2.5411x over previous
import functools

import jax
import jax.numpy as jnp
from jax import lax
from jax.experimental import pallas as pl
from jax.experimental.pallas import tpu as pltpu
from jax.experimental.pallas import tpu_sc as plsc

_N_E = 8192
_E_DIM = 32
_ROWS = 8192
_R = 256
_NB = _ROWS // _R


def _tc_body(x_ref, et_ref, ind_ref, loss_ref, colsum_ref, commit_ref):
    i = pl.program_id(0)
    xb = x_ref[...]
    nrm = jnp.sqrt(jnp.sum(xb * xb, axis=1, keepdims=True))
    xf = xb / jnp.maximum(nrm, 1e-12)
    d = lax.dot_general(xf, et_ref[...], (((1,), (0,)), ((), ())),
                        preferred_element_type=jnp.float32)
    minval = jnp.min(d, axis=1, keepdims=True)
    col = lax.broadcasted_iota(jnp.int32, d.shape, 1)
    idx = jnp.min(jnp.where(d == minval, col, jnp.int32(2**30)), axis=1)
    ind_ref[0, 0, :] = idx

    e = jnp.exp(d * 10.0)
    s = jnp.sum(e, axis=1, keepdims=True)
    pc = jnp.sum(e / s, axis=0, keepdims=True)
    cb = jnp.sum(2.0 - 2.0 * minval, axis=(0, 1), keepdims=True) / (
        float(_ROWS) * float(_E_DIM))

    @pl.when(i == 0)
    def _():
        colsum_ref[...] = pc
        commit_ref[...] = cb

    @pl.when(i > 0)
    def _():
        colsum_ref[...] += pc
        commit_ref[...] += cb

    @pl.when(i == _NB - 1)
    def _():
        ap = colsum_ref[...] * (1.0 / float(_ROWS))
        ent = -jnp.sum(ap * jnp.log(ap), axis=(0, 1), keepdims=True)
        loss_ref[...] = commit_ref[...] - ent


def _tc_call(x2d, embed_t):
    return pl.pallas_call(
        _tc_body,
        grid=(_NB,),
        in_specs=[
            pl.BlockSpec((_R, _E_DIM), lambda i: (i, 0)),
            pl.BlockSpec((_E_DIM, _N_E), lambda i: (0, 0)),
        ],
        out_specs=[
            pl.BlockSpec((1, 1, _R), lambda i: (i, 0, 0)),
            pl.BlockSpec((1, 1), lambda i: (0, 0)),
        ],
        out_shape=[
            jax.ShapeDtypeStruct((_NB, 1, _R), jnp.int32),
            jax.ShapeDtypeStruct((1, 1), jnp.float32),
        ],
        scratch_shapes=[
            pltpu.VMEM((1, _N_E), jnp.float32),
            pltpu.VMEM((1, 1), jnp.float32),
        ],
    )(x2d, embed_t)


def _sc_gather(embed, idx3):
    info = plsc.get_sparse_core_info()
    nc, ns = info.num_cores, info.num_subcores
    nw = nc * ns
    bpw = _ROWS // nw
    nchunk = bpw // 128
    mesh = plsc.VectorSubcoreMesh(core_axis_name="c", subcore_axis_name="s")

    @functools.partial(
        pl.kernel, mesh=mesh,
        compiler_params=pltpu.CompilerParams(use_tc_tiling_on_sc=False),
        out_type=jax.ShapeDtypeStruct((_ROWS, _E_DIM), jnp.float32),
        scratch_types=[
            pltpu.VMEM((nchunk, 128), jnp.int32),
            pltpu.VMEM((bpw, _E_DIM), jnp.float32),
            pltpu.SemaphoreType.DMA,
        ],
    )
    def k(table_hbm, idx_hbm, out_hbm, idx_v, rows_v, sem):
        wid = lax.axis_index("s") * nc + lax.axis_index("c")
        base = wid * bpw
        pltpu.sync_copy(idx_hbm.at[wid], idx_v)
        copies = [
            pltpu.async_copy(table_hbm.at[idx_v.at[j]],
                             rows_v.at[pl.ds(j * 128, 128)], sem)
            for j in range(nchunk)
        ]
        for c in copies:
            c.wait()
        pltpu.sync_copy(rows_v, out_hbm.at[pl.ds(base, bpw)])

    return k(embed, idx3)


def kernel(x, embed):
    x2d = x.reshape(_ROWS, _E_DIM)
    ind3, loss2 = _tc_call(x2d, embed.T)
    idx3 = ind3.reshape(32, 2, 128)
    z2d = _sc_gather(embed, idx3)
    z_q = z2d.reshape(x.shape)
    embed_ind = ind3.reshape(_ROWS)
    loss = loss2.reshape(())
    return z_q, embed_ind, loss


# speedup vs baseline: 2.9784x; 1.1721x over previous
import functools

import jax
import jax.numpy as jnp
from jax import lax
from jax.experimental import pallas as pl
from jax.experimental.pallas import tpu as pltpu
from jax.experimental.pallas import tpu_sc as plsc

_N_E = 8192
_E_DIM = 32
_ROWS = 8192
_R = 256
_NB = _ROWS // _R


def _tc_body(x_ref, et_ref, colf_ref, ind_ref, loss_ref, colsum_ref, commit_ref):
    i = pl.program_id(0)
    xb = x_ref[...]
    nrm = jnp.sqrt(jnp.sum(xb * xb, axis=1, keepdims=True))
    xf = xb / jnp.maximum(nrm, 1e-12)
    d = lax.dot_general(xf, et_ref[...], (((1,), (0,)), ((), ())),
                        preferred_element_type=jnp.float32)
    minval = jnp.min(d, axis=1, keepdims=True)
    idxf = jnp.min(jnp.where(d == minval, colf_ref[...], jnp.float32(2**24)),
                   axis=1)
    ind_ref[0, 0, :] = idxf.astype(jnp.int32)

    e = jnp.exp(d * 10.0)
    s = jnp.sum(e, axis=1, keepdims=True)
    r = (1.0 / s).reshape(1, _R)
    pc = lax.dot_general(r, e, (((1,), (0,)), ((), ())),
                         preferred_element_type=jnp.float32)
    cb = jnp.sum(2.0 - 2.0 * minval, axis=(0, 1), keepdims=True) / (
        float(_ROWS) * float(_E_DIM))

    @pl.when(i == 0)
    def _():
        colsum_ref[...] = pc
        commit_ref[...] = cb

    @pl.when(i > 0)
    def _():
        colsum_ref[...] += pc
        commit_ref[...] += cb

    @pl.when(i == _NB - 1)
    def _():
        ap = colsum_ref[...] * (1.0 / float(_ROWS))
        ent = -jnp.sum(ap * jnp.log(ap), axis=(0, 1), keepdims=True)
        loss_ref[...] = commit_ref[...] - ent


def _tc_call(x2d, embed_t):
    colf = lax.broadcasted_iota(jnp.float32, (1, _N_E), 1)
    return pl.pallas_call(
        _tc_body,
        grid=(_NB,),
        in_specs=[
            pl.BlockSpec((_R, _E_DIM), lambda i: (i, 0)),
            pl.BlockSpec((_E_DIM, _N_E), lambda i: (0, 0)),
            pl.BlockSpec((1, _N_E), lambda i: (0, 0)),
        ],
        out_specs=[
            pl.BlockSpec((1, 1, _R), lambda i: (i, 0, 0)),
            pl.BlockSpec((1, 1), lambda i: (0, 0)),
        ],
        out_shape=[
            jax.ShapeDtypeStruct((_NB, 1, _R), jnp.int32),
            jax.ShapeDtypeStruct((1, 1), jnp.float32),
        ],
        scratch_shapes=[
            pltpu.VMEM((1, _N_E), jnp.float32),
            pltpu.VMEM((1, 1), jnp.float32),
        ],
    )(x2d, embed_t, colf)


def _sc_gather(embed, idx3):
    info = plsc.get_sparse_core_info()
    nc, ns = info.num_cores, info.num_subcores
    nw = nc * ns
    bpw = _ROWS // nw
    nchunk = bpw // 128
    mesh = plsc.VectorSubcoreMesh(core_axis_name="c", subcore_axis_name="s")

    @functools.partial(
        pl.kernel, mesh=mesh,
        compiler_params=pltpu.CompilerParams(use_tc_tiling_on_sc=False),
        out_type=jax.ShapeDtypeStruct((_ROWS, _E_DIM), jnp.float32),
        scratch_types=[
            pltpu.VMEM((nchunk, 128), jnp.int32),
            pltpu.VMEM((bpw, _E_DIM), jnp.float32),
            pltpu.SemaphoreType.DMA,
        ],
    )
    def k(table_hbm, idx_hbm, out_hbm, idx_v, rows_v, sem):
        wid = lax.axis_index("s") * nc + lax.axis_index("c")
        base = wid * bpw
        pltpu.sync_copy(idx_hbm.at[wid], idx_v)
        copies = [
            pltpu.async_copy(table_hbm.at[idx_v.at[j]],
                             rows_v.at[pl.ds(j * 128, 128)], sem)
            for j in range(nchunk)
        ]
        for c in copies:
            c.wait()
        pltpu.sync_copy(rows_v, out_hbm.at[pl.ds(base, bpw)])

    return k(embed, idx3)


def kernel(x, embed):
    x2d = x.reshape(_ROWS, _E_DIM)
    ind3, loss2 = _tc_call(x2d, embed.T)
    idx3 = ind3.reshape(32, 2, 128)
    z2d = _sc_gather(embed, idx3)
    z_q = z2d.reshape(x.shape)
    embed_ind = ind3.reshape(_ROWS)
    loss = loss2.reshape(())
    return z_q, embed_ind, loss


# speedup vs baseline: 3.2455x; 1.0897x over previous
import functools

import jax
import jax.numpy as jnp
from jax import lax
from jax.experimental import pallas as pl
from jax.experimental.pallas import tpu as pltpu
from jax.experimental.pallas import tpu_sc as plsc

_N_E = 8192
_E_DIM = 32
_ROWS = 8192
_R = 512
_NB = _ROWS // _R


def _tc_body(x_ref, et_ref, colf_ref, ind_ref, loss_ref, colsum_ref, commit_ref):
    i = pl.program_id(0)
    xb = x_ref[...]
    nrm = jnp.sqrt(jnp.sum(xb * xb, axis=1, keepdims=True))
    xf = xb / jnp.maximum(nrm, 1e-12)
    d = lax.dot_general(xf, et_ref[...], (((1,), (0,)), ((), ())),
                        preferred_element_type=jnp.float32)
    minval = jnp.min(d, axis=1, keepdims=True)
    idxf = jnp.min(jnp.where(d == minval, colf_ref[...], jnp.float32(2**24)),
                   axis=1)
    ind_ref[0, 0, :] = idxf.astype(jnp.int32)

    e = jnp.exp(d * 10.0)
    s = jnp.sum(e, axis=1, keepdims=True)
    r = (1.0 / s).reshape(1, _R)
    pc = lax.dot_general(r, e, (((1,), (0,)), ((), ())),
                         preferred_element_type=jnp.float32)
    cb = jnp.sum(2.0 - 2.0 * minval, axis=(0, 1), keepdims=True) / (
        float(_ROWS) * float(_E_DIM))

    @pl.when(i == 0)
    def _():
        colsum_ref[...] = pc
        commit_ref[...] = cb

    @pl.when(i > 0)
    def _():
        colsum_ref[...] += pc
        commit_ref[...] += cb

    @pl.when(i == _NB - 1)
    def _():
        ap = colsum_ref[...] * (1.0 / float(_ROWS))
        ent = -jnp.sum(ap * jnp.log(ap), axis=(0, 1), keepdims=True)
        loss_ref[...] = commit_ref[...] - ent


def _tc_call(x2d, embed_t):
    colf = lax.broadcasted_iota(jnp.float32, (1, _N_E), 1)
    return pl.pallas_call(
        _tc_body,
        grid=(_NB,),
        in_specs=[
            pl.BlockSpec((_R, _E_DIM), lambda i: (i, 0)),
            pl.BlockSpec((_E_DIM, _N_E), lambda i: (0, 0)),
            pl.BlockSpec((1, _N_E), lambda i: (0, 0)),
        ],
        out_specs=[
            pl.BlockSpec((1, 1, _R), lambda i: (i, 0, 0)),
            pl.BlockSpec((1, 1), lambda i: (0, 0)),
        ],
        out_shape=[
            jax.ShapeDtypeStruct((_NB, 1, _R), jnp.int32),
            jax.ShapeDtypeStruct((1, 1), jnp.float32),
        ],
        scratch_shapes=[
            pltpu.VMEM((1, _N_E), jnp.float32),
            pltpu.VMEM((1, 1), jnp.float32),
        ],
    )(x2d, embed_t, colf)


def _sc_gather(embed, idx3):
    info = plsc.get_sparse_core_info()
    nc, ns = info.num_cores, info.num_subcores
    nw = nc * ns
    bpw = _ROWS // nw
    nchunk = bpw // 128
    mesh = plsc.VectorSubcoreMesh(core_axis_name="c", subcore_axis_name="s")

    @functools.partial(
        pl.kernel, mesh=mesh,
        compiler_params=pltpu.CompilerParams(use_tc_tiling_on_sc=False),
        out_type=jax.ShapeDtypeStruct((_ROWS, _E_DIM), jnp.float32),
        scratch_types=[
            pltpu.VMEM((nchunk, 128), jnp.int32),
            pltpu.VMEM((bpw, _E_DIM), jnp.float32),
            pltpu.SemaphoreType.DMA,
        ],
    )
    def k(table_hbm, idx_hbm, out_hbm, idx_v, rows_v, sem):
        wid = lax.axis_index("s") * nc + lax.axis_index("c")
        base = wid * bpw
        pltpu.sync_copy(idx_hbm.at[wid], idx_v)
        copies = [
            pltpu.async_copy(table_hbm.at[idx_v.at[j]],
                             rows_v.at[pl.ds(j * 128, 128)], sem)
            for j in range(nchunk)
        ]
        for c in copies:
            c.wait()
        pltpu.sync_copy(rows_v, out_hbm.at[pl.ds(base, bpw)])

    return k(embed, idx3)


def kernel(x, embed):
    x2d = x.reshape(_ROWS, _E_DIM)
    ind3, loss2 = _tc_call(x2d, embed.T)
    idx3 = ind3.reshape(32, 2, 128)
    z2d = _sc_gather(embed, idx3)
    z_q = z2d.reshape(x.shape)
    embed_ind = ind3.reshape(_ROWS)
    loss = loss2.reshape(())
    return z_q, embed_ind, loss


# speedup vs baseline: 3.2931x; 1.0147x over previous
import functools

import jax
import jax.numpy as jnp
from jax import lax
from jax.experimental import pallas as pl
from jax.experimental.pallas import tpu as pltpu
from jax.experimental.pallas import tpu_sc as plsc

_N_E = 8192
_E_DIM = 32
_ROWS = 8192
_R = 1024
_NB = _ROWS // _R


def _tc_body(x_ref, et_ref, colf_ref, ind_ref, loss_ref, colsum_ref, commit_ref):
    i = pl.program_id(0)
    xb = x_ref[...]
    nrm = jnp.sqrt(jnp.sum(xb * xb, axis=1, keepdims=True))
    xf = xb / jnp.maximum(nrm, 1e-12)
    d = lax.dot_general(xf, et_ref[...], (((1,), (0,)), ((), ())),
                        preferred_element_type=jnp.float32)
    minval = jnp.min(d, axis=1, keepdims=True)
    idxf = jnp.min(jnp.where(d == minval, colf_ref[...], jnp.float32(2**24)),
                   axis=1)
    ind_ref[0, 0, :] = idxf.astype(jnp.int32)

    e16 = jnp.exp(d * 10.0).astype(jnp.bfloat16)
    s = jnp.sum(e16, axis=1, keepdims=True, dtype=jnp.float32)
    r16 = (1.0 / s).astype(jnp.bfloat16).reshape(1, _R)
    pc = lax.dot_general(r16, e16, (((1,), (0,)), ((), ())),
                         preferred_element_type=jnp.float32)
    cb = jnp.sum(2.0 - 2.0 * minval, axis=(0, 1), keepdims=True) / (
        float(_ROWS) * float(_E_DIM))

    @pl.when(i == 0)
    def _():
        colsum_ref[...] = pc
        commit_ref[...] = cb

    @pl.when(i > 0)
    def _():
        colsum_ref[...] += pc
        commit_ref[...] += cb

    @pl.when(i == _NB - 1)
    def _():
        ap = colsum_ref[...] * (1.0 / float(_ROWS))
        ent = -jnp.sum(ap * jnp.log(ap), axis=(0, 1), keepdims=True)
        loss_ref[...] = commit_ref[...] - ent


def _tc_call(x2d, embed_t):
    colf = lax.broadcasted_iota(jnp.float32, (1, _N_E), 1)
    return pl.pallas_call(
        _tc_body,
        grid=(_NB,),
        in_specs=[
            pl.BlockSpec((_R, _E_DIM), lambda i: (i, 0)),
            pl.BlockSpec((_E_DIM, _N_E), lambda i: (0, 0)),
            pl.BlockSpec((1, _N_E), lambda i: (0, 0)),
        ],
        out_specs=[
            pl.BlockSpec((1, 1, _R), lambda i: (i, 0, 0)),
            pl.BlockSpec((1, 1), lambda i: (0, 0)),
        ],
        out_shape=[
            jax.ShapeDtypeStruct((_NB, 1, _R), jnp.int32),
            jax.ShapeDtypeStruct((1, 1), jnp.float32),
        ],
        scratch_shapes=[
            pltpu.VMEM((1, _N_E), jnp.float32),
            pltpu.VMEM((1, 1), jnp.float32),
        ],
    )(x2d, embed_t, colf)


def _sc_gather(embed, idx3):
    info = plsc.get_sparse_core_info()
    nc, ns = info.num_cores, info.num_subcores
    nw = nc * ns
    bpw = _ROWS // nw
    nchunk = bpw // 128
    mesh = plsc.VectorSubcoreMesh(core_axis_name="c", subcore_axis_name="s")

    @functools.partial(
        pl.kernel, mesh=mesh,
        compiler_params=pltpu.CompilerParams(use_tc_tiling_on_sc=False),
        out_type=jax.ShapeDtypeStruct((_ROWS, _E_DIM), jnp.float32),
        scratch_types=[
            pltpu.VMEM((nchunk, 128), jnp.int32),
            pltpu.VMEM((bpw, _E_DIM), jnp.float32),
            pltpu.SemaphoreType.DMA,
        ],
    )
    def k(table_hbm, idx_hbm, out_hbm, idx_v, rows_v, sem):
        wid = lax.axis_index("s") * nc + lax.axis_index("c")
        base = wid * bpw
        pltpu.sync_copy(idx_hbm.at[wid], idx_v)
        copies = [
            pltpu.async_copy(table_hbm.at[idx_v.at[j]],
                             rows_v.at[pl.ds(j * 128, 128)], sem)
            for j in range(nchunk)
        ]
        for c in copies:
            c.wait()
        pltpu.sync_copy(rows_v, out_hbm.at[pl.ds(base, bpw)])

    return k(embed, idx3)


def kernel(x, embed):
    x2d = x.reshape(_ROWS, _E_DIM)
    ind3, loss2 = _tc_call(x2d, embed.T)
    idx3 = ind3.reshape(32, 2, 128)
    z2d = _sc_gather(embed, idx3)
    z_q = z2d.reshape(x.shape)
    embed_ind = ind3.reshape(_ROWS)
    loss = loss2.reshape(())
    return z_q, embed_ind, loss


# speedup vs baseline: 3.4672x; 1.0529x over previous
import functools

import jax
import jax.numpy as jnp
from jax import lax
from jax.experimental import pallas as pl
from jax.experimental.pallas import tpu as pltpu
from jax.experimental.pallas import tpu_sc as plsc

_N_E = 8192
_E_DIM = 32
_ROWS = 8192
_R = 1024
_NB = _ROWS // _R


def _tc_body(x_ref, et_ref, colf_ref, ones_ref, ind_ref, loss_ref, colsum_ref,
             commit_ref):
    i = pl.program_id(0)
    xb = x_ref[...]
    nrm = jnp.sqrt(jnp.sum(xb * xb, axis=1, keepdims=True))
    xf = xb / jnp.maximum(nrm, 1e-12)
    d = lax.dot_general(xf, et_ref[...], (((1,), (0,)), ((), ())),
                        preferred_element_type=jnp.float32)
    minval = jnp.min(d, axis=1, keepdims=True)
    idxf = jnp.min(jnp.where(d == minval, colf_ref[...], jnp.float32(2**24)),
                   axis=1)
    ind_ref[0, 0, :] = idxf.astype(jnp.int32)

    e16 = jnp.exp2(d * (10.0 / 0.6931471805599453)).astype(jnp.bfloat16)
    s = jnp.sum(e16, axis=1, keepdims=True, dtype=jnp.float32)
    r16 = (1.0 / s).astype(jnp.bfloat16).reshape(1, _R)
    pc = lax.dot_general(r16, e16, (((1,), (0,)), ((), ())),
                         preferred_element_type=jnp.float32)
    cb = jnp.sum(2.0 - 2.0 * minval, axis=(0, 1), keepdims=True) / (
        float(_ROWS) * float(_E_DIM))

    @pl.when(i == 0)
    def _():
        colsum_ref[...] = pc
        commit_ref[...] = cb

    @pl.when(i > 0)
    def _():
        colsum_ref[...] += pc
        commit_ref[...] += cb

    @pl.when(i == _NB - 1)
    def _():
        ap = colsum_ref[...] * (1.0 / float(_ROWS))
        ent = -jnp.sum(ap * jnp.log(ap), axis=(0, 1), keepdims=True)
        loss_ref[...] = commit_ref[...] - ent


def _tc_call(x2d, embed_t):
    colf = lax.broadcasted_iota(jnp.float32, (1, _N_E), 1)
    ones16 = jnp.ones((_N_E, 1), dtype=jnp.bfloat16)
    return pl.pallas_call(
        _tc_body,
        grid=(_NB,),
        in_specs=[
            pl.BlockSpec((_R, _E_DIM), lambda i: (i, 0)),
            pl.BlockSpec((_E_DIM, _N_E), lambda i: (0, 0)),
            pl.BlockSpec((1, _N_E), lambda i: (0, 0)),
            pl.BlockSpec((_N_E, 1), lambda i: (0, 0)),
        ],
        out_specs=[
            pl.BlockSpec((1, 1, _R), lambda i: (i, 0, 0)),
            pl.BlockSpec((1, 1), lambda i: (0, 0)),
        ],
        out_shape=[
            jax.ShapeDtypeStruct((_NB, 1, _R), jnp.int32),
            jax.ShapeDtypeStruct((1, 1), jnp.float32),
        ],
        scratch_shapes=[
            pltpu.VMEM((1, _N_E), jnp.float32),
            pltpu.VMEM((1, 1), jnp.float32),
        ],
    )(x2d, embed_t, colf, ones16)


def _sc_gather(embed, idx3):
    info = plsc.get_sparse_core_info()
    nc, ns = info.num_cores, info.num_subcores
    nw = nc * ns
    bpw = _ROWS // nw
    nchunk = bpw // 128
    mesh = plsc.VectorSubcoreMesh(core_axis_name="c", subcore_axis_name="s")

    @functools.partial(
        pl.kernel, mesh=mesh,
        compiler_params=pltpu.CompilerParams(use_tc_tiling_on_sc=False),
        out_type=jax.ShapeDtypeStruct((_ROWS, _E_DIM), jnp.float32),
        scratch_types=[
            pltpu.VMEM((nchunk, 128), jnp.int32),
            pltpu.VMEM((bpw, _E_DIM), jnp.float32),
            pltpu.SemaphoreType.DMA,
        ],
    )
    def k(table_hbm, idx_hbm, out_hbm, idx_v, rows_v, sem):
        wid = lax.axis_index("s") * nc + lax.axis_index("c")
        base = wid * bpw
        pltpu.sync_copy(idx_hbm.at[wid], idx_v)
        copies = [
            pltpu.async_copy(table_hbm.at[idx_v.at[j]],
                             rows_v.at[pl.ds(j * 128, 128)], sem)
            for j in range(nchunk)
        ]
        for c in copies:
            c.wait()
        pltpu.sync_copy(rows_v, out_hbm.at[pl.ds(base, bpw)])

    return k(embed, idx3)


def kernel(x, embed):
    x2d = x.reshape(_ROWS, _E_DIM)
    ind3, loss2 = _tc_call(x2d, embed.T)
    idx3 = ind3.reshape(32, 2, 128)
    z2d = _sc_gather(embed, idx3)
    z_q = z2d.reshape(x.shape)
    embed_ind = ind3.reshape(_ROWS)
    loss = loss2.reshape(())
    return z_q, embed_ind, loss


# speedup vs baseline: 3.5263x; 1.0171x over previous
import functools

import jax
import jax.numpy as jnp
from jax import lax
from jax.experimental import pallas as pl
from jax.experimental.pallas import tpu as pltpu
from jax.experimental.pallas import tpu_sc as plsc

_N_E = 8192
_E_DIM = 32
_ROWS = 8192
_R = 1024
_NB = _ROWS // _R


def _tc_body(x_ref, et_ref, colf_ref, ind_ref, loss_ref, colsum_ref,
             commit_ref):
    i = pl.program_id(0)
    xb = x_ref[...]
    nrm = jnp.sqrt(jnp.sum(xb * xb, axis=1, keepdims=True))
    xf = xb / jnp.maximum(nrm, 1e-12)
    d = lax.dot_general(xf, et_ref[...], (((1,), (0,)), ((), ())),
                        preferred_element_type=jnp.float32)
    minval = jnp.min(d, axis=1, keepdims=True)
    idxf = jnp.min(jnp.where(d == minval, colf_ref[...], jnp.float32(2**24)),
                   axis=1)
    ind_ref[0, 0, :] = idxf.astype(jnp.int32)

    e16 = jnp.exp2(d * (10.0 / 0.6931471805599453)).astype(jnp.bfloat16)
    s = jnp.sum(e16, axis=1, keepdims=True, dtype=jnp.float32)
    r16 = (1.0 / s).astype(jnp.bfloat16).reshape(1, _R)
    pc = lax.dot_general(r16, e16, (((1,), (0,)), ((), ())),
                         preferred_element_type=jnp.float32)
    cb = jnp.sum(2.0 - 2.0 * minval, axis=(0, 1), keepdims=True) / (
        float(_ROWS) * float(_E_DIM))

    @pl.when(i == 0)
    def _():
        colsum_ref[...] = pc
        commit_ref[...] = cb

    @pl.when(i > 0)
    def _():
        colsum_ref[...] += pc
        commit_ref[...] += cb

    @pl.when(i == _NB - 1)
    def _():
        ap = colsum_ref[...] * (1.0 / float(_ROWS))
        ent = -jnp.sum(ap * jnp.log(ap), axis=(0, 1), keepdims=True)
        loss_ref[...] = commit_ref[...] - ent


def _tc_call(x2d, embed_t):
    colf = lax.broadcasted_iota(jnp.float32, (1, _N_E), 1)
    return pl.pallas_call(
        _tc_body,
        grid=(_NB,),
        in_specs=[
            pl.BlockSpec((_R, _E_DIM), lambda i: (i, 0)),
            pl.BlockSpec((_E_DIM, _N_E), lambda i: (0, 0)),
            pl.BlockSpec((1, _N_E), lambda i: (0, 0)),
        ],
        out_specs=[
            pl.BlockSpec((1, 1, _R), lambda i: (i, 0, 0)),
            pl.BlockSpec((1, 1), lambda i: (0, 0)),
        ],
        out_shape=[
            jax.ShapeDtypeStruct((_NB, 1, _R), jnp.int32),
            jax.ShapeDtypeStruct((1, 1), jnp.float32),
        ],
        scratch_shapes=[
            pltpu.VMEM((1, _N_E), jnp.float32),
            pltpu.VMEM((1, 1), jnp.float32),
        ],
    )(x2d, embed_t, colf)


def _sc_gather(embed, idx3):
    info = plsc.get_sparse_core_info()
    nc, ns = info.num_cores, info.num_subcores
    nw = nc * ns
    bpw = _ROWS // nw
    nchunk = bpw // 128
    mesh = plsc.VectorSubcoreMesh(core_axis_name="c", subcore_axis_name="s")

    @functools.partial(
        pl.kernel, mesh=mesh,
        compiler_params=pltpu.CompilerParams(use_tc_tiling_on_sc=False),
        out_type=jax.ShapeDtypeStruct((_ROWS, _E_DIM), jnp.float32),
        scratch_types=[
            pltpu.VMEM((nchunk, 128), jnp.int32),
            pltpu.VMEM((bpw, _E_DIM), jnp.float32),
            pltpu.SemaphoreType.DMA,
        ],
    )
    def k(table_hbm, idx_hbm, out_hbm, idx_v, rows_v, sem):
        wid = lax.axis_index("s") * nc + lax.axis_index("c")
        base = wid * bpw
        pltpu.sync_copy(idx_hbm.at[wid], idx_v)
        copies = [
            pltpu.async_copy(table_hbm.at[idx_v.at[j]],
                             rows_v.at[pl.ds(j * 128, 128)], sem)
            for j in range(nchunk)
        ]
        for c in copies:
            c.wait()
        pltpu.sync_copy(rows_v, out_hbm.at[pl.ds(base, bpw)])

    return k(embed, idx3)


def kernel(x, embed):
    x2d = x.reshape(_ROWS, _E_DIM)
    ind3, loss2 = _tc_call(x2d, embed.T)
    idx3 = ind3.reshape(32, 2, 128)
    z2d = _sc_gather(embed, idx3)
    z_q = z2d.reshape(x.shape)
    embed_ind = ind3.reshape(_ROWS)
    loss = loss2.reshape(())
    return z_q, embed_ind, loss
